# Initial kernel scaffold; baseline (speedup 1.0000x reference)
import jax
import jax.numpy as jnp
from jax.experimental import pallas as pl


def kernel(x, neighbors):
    raise NotImplementedError("write your pallas kernel here")


# speedup vs baseline: 1.1465x; 1.1465x over previous
import functools

import jax
import jax.numpy as jnp
from jax import lax
from jax.experimental import pallas as pl
from jax.experimental.pallas import tpu as pltpu
from jax.experimental.pallas import tpu_sc as plsc

N, K, D = 10000, 32, 128
L = 16
NC, NS = 2, 16
NW = NC * NS
G = 4
GK = G * K
NBW = 320
NPAD = NW * NBW
NCH = NBW // G
DV = D // L

_mesh = plsc.VectorSubcoreMesh(core_axis_name="c", subcore_axis_name="s")


@functools.partial(
    pl.kernel,
    out_type=jax.ShapeDtypeStruct((NPAD, D), jnp.float32),
    mesh=_mesh,
    scratch_types=[
        pltpu.VMEM((NBW * K,), jnp.int32),
        pltpu.VMEM((GK, D), jnp.float32),
        pltpu.VMEM((NBW, D), jnp.float32),
        pltpu.SemaphoreType.DMA,
    ],
)
def _pool_kernel(x_hbm, nbr_hbm, out_hbm, idx_v, rows_v, acc_v, sem):
    wid = lax.axis_index("s") * NC + lax.axis_index("c")
    base_node = wid * NBW
    pltpu.sync_copy(nbr_hbm.at[pl.ds(base_node * K, NBW * K)], idx_v)

    def chunk_body(c, carry):
        pltpu.async_copy(
            x_hbm.at[idx_v.at[pl.ds(c * GK, GK)]], rows_v, sem
        ).wait()
        for g in range(G):
            row0 = g * K
            accs = [rows_v[row0, pl.ds(d * L, L)] for d in range(DV)]
            for k in range(1, K):
                accs = [
                    jnp.maximum(a, rows_v[row0 + k, pl.ds(d * L, L)])
                    for d, a in enumerate(accs)
                ]
            node = c * G + g
            for d in range(DV):
                acc_v[node, pl.ds(d * L, L)] = accs[d]
        return carry

    lax.fori_loop(0, NCH, chunk_body, 0)
    pltpu.sync_copy(acc_v, out_hbm.at[pl.ds(base_node, NBW)])


def kernel(x, neighbors):
    nbr_flat = jnp.concatenate(
        [neighbors.reshape(-1), jnp.zeros(((NPAD - N) * K,), jnp.int32)]
    )
    out = _pool_kernel(x, nbr_flat)
    return out[:N]


# speedup vs baseline: 1.4897x; 1.2993x over previous
import functools

import jax
import jax.numpy as jnp
from jax import lax
from jax.experimental import pallas as pl
from jax.experimental.pallas import tpu as pltpu
from jax.experimental.pallas import tpu_sc as plsc

N, K, D = 10000, 32, 128
L = 16
NC, NS = 2, 16
NW = NC * NS
G = 4
GK = G * K
NBW = 320
NPAD = NW * NBW
NCH = NBW // G
DV = D // L

_mesh = plsc.VectorSubcoreMesh(core_axis_name="c", subcore_axis_name="s")


NBUF = 2


@functools.partial(
    pl.kernel,
    out_type=jax.ShapeDtypeStruct((NPAD, D), jnp.float32),
    mesh=_mesh,
    scratch_types=[
        pltpu.VMEM((NBW * K,), jnp.int32),
        pltpu.VMEM((NBUF, GK, D), jnp.float32),
        pltpu.VMEM((NBW, D), jnp.float32),
        pltpu.SemaphoreType.DMA,
        pltpu.SemaphoreType.DMA,
    ],
)
def _pool_kernel(x_hbm, nbr_hbm, out_hbm, idx_v, rows_v, acc_v, sem0, sem1):
    wid = lax.axis_index("s") * NC + lax.axis_index("c")
    base_node = wid * NBW
    sems = [sem0, sem1]
    pltpu.sync_copy(nbr_hbm.at[pl.ds(base_node * K, NBW * K)], idx_v)

    def gather(c, b):
        pltpu.make_async_copy(
            x_hbm.at[idx_v.at[pl.ds(c * GK, GK)]], rows_v.at[b], sems[b]
        ).start()

    for b in range(NBUF):
        gather(b, b)

    def step_body(i, carry):
        for b in range(NBUF):
            c = i * NBUF + b
            pltpu.make_async_copy(
                x_hbm.at[idx_v.at[pl.ds(c * GK, GK)]], rows_v.at[b], sems[b]
            ).wait()
            for g in range(G):
                row0 = g * K
                accs = [rows_v[b, row0, pl.ds(d * L, L)] for d in range(DV)]
                for k in range(1, K):
                    accs = [
                        jnp.maximum(a, rows_v[b, row0 + k, pl.ds(d * L, L)])
                        for d, a in enumerate(accs)
                    ]
                node = c * G + g
                for d in range(DV):
                    acc_v[node, pl.ds(d * L, L)] = accs[d]
            nxt = c + NBUF

            @pl.when(nxt < NCH)
            def _():
                gather(nxt, b)

        return carry

    lax.fori_loop(0, NCH // NBUF, step_body, 0)
    pltpu.sync_copy(acc_v, out_hbm.at[pl.ds(base_node, NBW)])


def kernel(x, neighbors):
    nbr_flat = jnp.concatenate(
        [neighbors.reshape(-1), jnp.zeros(((NPAD - N) * K,), jnp.int32)]
    )
    out = _pool_kernel(x, nbr_flat)
    return out[:N]
